# Initial kernel scaffold; baseline (speedup 1.0000x reference)
import jax
import jax.numpy as jnp
from jax.experimental import pallas as pl


def kernel(idx, table):
    raise NotImplementedError("write your pallas kernel here")


# speedup vs baseline: 1.2320x; 1.2320x over previous
import functools

import jax
import jax.numpy as jnp
from jax import lax
from jax.experimental import pallas as pl
from jax.experimental.pallas import tpu as pltpu
from jax.experimental.pallas import tpu_sc as plsc

D = 128
NC, NS = 2, 16
NW = NC * NS
CHUNK = 128
GRP = 8


def _make_gather(B):
    assert B % (NW * GRP * CHUNK) == 0
    b_per_w = B // NW
    rows_per_w = b_per_w // CHUNK
    n_grp = rows_per_w // GRP
    mesh = plsc.VectorSubcoreMesh(core_axis_name="c", subcore_axis_name="s")

    @functools.partial(
        pl.kernel,
        out_type=jax.ShapeDtypeStruct((B, D), jnp.float32),
        mesh=mesh,
        scratch_types=[
            pltpu.VMEM((GRP, CHUNK), jnp.int32),
            pltpu.VMEM((CHUNK, D), jnp.float32),
            pltpu.SemaphoreType.DMA,
        ],
    )
    def gather_kernel(idx_hbm, table_hbm, out_hbm, idx_v, rows_v, sem):
        wid = lax.axis_index("s") * NC + lax.axis_index("c")
        row_base = wid * rows_per_w

        def grp_body(g, carry):
            row0 = row_base + g * GRP
            pltpu.sync_copy(idx_hbm.at[pl.ds(row0, GRP)], idx_v)
            for j in range(GRP):
                pltpu.async_copy(table_hbm.at[idx_v.at[j]], rows_v, sem).wait()
                pltpu.sync_copy(
                    rows_v, out_hbm.at[pl.ds((row0 + j) * CHUNK, CHUNK)])
            return carry

        lax.fori_loop(0, n_grp, grp_body, 0)

    return gather_kernel


def kernel(idx, table):
    B_rows, H = idx.shape
    B = B_rows * H
    idx2d = idx.reshape(B // CHUNK, CHUNK)
    out = _make_gather(B)(idx2d, table)
    return out.reshape(B_rows, H, D)


# speedup vs baseline: 1.8674x; 1.5158x over previous
import functools

import jax
import jax.numpy as jnp
from jax import lax
from jax.experimental import pallas as pl
from jax.experimental.pallas import tpu as pltpu
from jax.experimental.pallas import tpu_sc as plsc

D = 128
NC, NS = 2, 16
NW = NC * NS
CHUNK = 128
KCH = 2
SLOT = KCH * CHUNK
NBUF = 2


def _make_gather(B):
    assert B % (NW * NBUF * SLOT) == 0
    b_per_w = B // NW
    rows_per_w = b_per_w // CHUNK
    n_slots = rows_per_w // KCH
    n_pairs = n_slots // NBUF
    mesh = plsc.VectorSubcoreMesh(core_axis_name="c", subcore_axis_name="s")

    @functools.partial(
        pl.kernel,
        out_type=jax.ShapeDtypeStruct((B, D), jnp.float32),
        mesh=mesh,
        scratch_types=[
            pltpu.VMEM((NBUF, KCH, CHUNK), jnp.int32),
            pltpu.VMEM((NBUF, SLOT, D), jnp.float32),
            pltpu.SemaphoreType.DMA((NBUF,)),
            pltpu.SemaphoreType.DMA((NBUF,)),
        ],
    )
    def gather_kernel(idx_hbm, table_hbm, out_hbm, idx_v, rows_v, sem_g, sem_w):
        wid = lax.axis_index("s") * NC + lax.axis_index("c")
        row_base = wid * rows_per_w

        def fire(s, b):
            row0 = row_base + s * KCH
            pltpu.sync_copy(idx_hbm.at[pl.ds(row0, KCH)], idx_v.at[b])
            for j in range(KCH):
                pltpu.async_copy(
                    table_hbm.at[idx_v.at[b, j]],
                    rows_v.at[b, pl.ds(j * CHUNK, CHUNK)],
                    sem_g.at[b])

        def wait_gathers(s, b):
            for j in range(KCH):
                pltpu.make_async_copy(
                    table_hbm.at[idx_v.at[b, j]],
                    rows_v.at[b, pl.ds(j * CHUNK, CHUNK)],
                    sem_g.at[b]).wait()

        def out_slice(s):
            return out_hbm.at[pl.ds((row_base + s * KCH) * CHUNK, SLOT)]

        for b in range(NBUF):
            fire(b, b)

        def body(g, carry):
            for b in range(NBUF):
                s = NBUF * g + b
                wait_gathers(s, b)
                pltpu.async_copy(rows_v.at[b], out_slice(s), sem_w.at[b])
                pltpu.make_async_copy(rows_v.at[b], out_slice(s),
                                      sem_w.at[b]).wait()

                @pl.when(s + NBUF < n_slots)
                def _():
                    fire(s + NBUF, b)
            return carry

        lax.fori_loop(0, n_pairs, body, 0)

    return gather_kernel


def kernel(idx, table):
    B_rows, H = idx.shape
    B = B_rows * H
    idx2d = idx.reshape(B // CHUNK, CHUNK)
    out = _make_gather(B)(idx2d, table)
    return out.reshape(B_rows, H, D)


# speedup vs baseline: 1.8984x; 1.0166x over previous
import functools

import jax
import jax.numpy as jnp
from jax import lax
from jax.experimental import pallas as pl
from jax.experimental.pallas import tpu as pltpu
from jax.experimental.pallas import tpu_sc as plsc

D = 128
NC, NS = 2, 16
NW = NC * NS
CHUNK = 128
KCH = 2
SLOT = KCH * CHUNK
NBUF = 2
SPB = 16
ROWS_PER_BLK = SPB * KCH


def _make_gather(B):
    assert B % (NW * NBUF * SLOT) == 0
    b_per_w = B // NW
    rows_per_w = b_per_w // CHUNK
    n_slots = rows_per_w // KCH
    n_pairs = n_slots // NBUF
    n_blocks = n_slots // SPB
    assert n_slots % SPB == 0
    mesh = plsc.VectorSubcoreMesh(core_axis_name="c", subcore_axis_name="s")

    @functools.partial(
        pl.kernel,
        out_type=jax.ShapeDtypeStruct((B, D), jnp.float32),
        mesh=mesh,
        scratch_types=[
            pltpu.VMEM((2, ROWS_PER_BLK, CHUNK), jnp.int32),
            pltpu.VMEM((NBUF, SLOT, D), jnp.float32),
            pltpu.SemaphoreType.DMA((NBUF,)),
            pltpu.SemaphoreType.DMA((NBUF,)),
            pltpu.SemaphoreType.DMA,
        ],
    )
    def gather_kernel(idx_hbm, table_hbm, out_hbm, idx_blk, rows_v,
                      sem_g, sem_w, sem_i):
        wid = lax.axis_index("s") * NC + lax.axis_index("c")
        row_base = wid * rows_per_w

        def blk_copy(kb, bsel):
            return pltpu.make_async_copy(
                idx_hbm.at[pl.ds(row_base + kb * ROWS_PER_BLK, ROWS_PER_BLK)],
                idx_blk.at[bsel], sem_i)

        def gath_copy(s, b, j):
            bsel = lax.rem(s // SPB, 2)
            r = lax.rem(s, SPB) * KCH + j
            return pltpu.make_async_copy(
                table_hbm.at[idx_blk.at[bsel, r]],
                rows_v.at[b, pl.ds(j * CHUNK, CHUNK)], sem_g.at[b])

        def out_copy(s, b):
            return pltpu.make_async_copy(
                rows_v.at[b],
                out_hbm.at[pl.ds((row_base + s * KCH) * CHUNK, SLOT)],
                sem_w.at[b])

        pltpu.sync_copy(idx_hbm.at[pl.ds(row_base, ROWS_PER_BLK)],
                        idx_blk.at[0])
        for b in range(NBUF):
            for j in range(KCH):
                gath_copy(b, b, j).start()

        def body(g, carry):
            for b in range(NBUF):
                s = NBUF * g + b
                for j in range(KCH):
                    gath_copy(s, b, j).wait()
                out_copy(s, b).start()
                out_copy(s, b).wait()
                sf = s + NBUF

                @pl.when(sf < n_slots)
                def _():
                    @pl.when(lax.rem(sf, SPB) == 0)
                    def _():
                        blk_copy(sf // SPB, lax.rem(sf // SPB, 2)).wait()

                    for j in range(KCH):
                        gath_copy(sf, b, j).start()

                @pl.when((lax.rem(s, SPB) == 0) & (s // SPB + 1 < n_blocks))
                def _():
                    kb = s // SPB + 1
                    blk_copy(kb, lax.rem(kb, 2)).start()
            return carry

        lax.fori_loop(0, n_pairs, body, 0)

    return gather_kernel


def kernel(idx, table):
    B_rows, H = idx.shape
    B = B_rows * H
    idx2d = idx.reshape(B // CHUNK, CHUNK)
    out = _make_gather(B)(idx2d, table)
    return out.reshape(B_rows, H, D)


# speedup vs baseline: 1.9035x; 1.0027x over previous
import functools

import jax
import jax.numpy as jnp
from jax import lax
from jax.experimental import pallas as pl
from jax.experimental.pallas import tpu as pltpu
from jax.experimental.pallas import tpu_sc as plsc

D = 128
NC, NS = 2, 16
NW = NC * NS
CHUNK = 128
NBUF = 4
SPB = 32


def _make_gather(B):
    assert B % (NW * CHUNK) == 0
    b_per_w = B // NW
    n_slots = b_per_w // CHUNK
    assert n_slots % SPB == 0 and n_slots % NBUF == 0
    n_quads = n_slots // NBUF
    n_blocks = n_slots // SPB
    mesh = plsc.VectorSubcoreMesh(core_axis_name="c", subcore_axis_name="s")

    @functools.partial(
        pl.kernel,
        out_type=jax.ShapeDtypeStruct((B, D), jnp.float32),
        mesh=mesh,
        scratch_types=[
            pltpu.VMEM((2, SPB, CHUNK), jnp.int32),
            pltpu.VMEM((NBUF, CHUNK, D), jnp.float32),
            pltpu.SemaphoreType.DMA((NBUF,)),
            pltpu.SemaphoreType.DMA((NBUF,)),
            pltpu.SemaphoreType.DMA,
        ],
    )
    def gather_kernel(idx_hbm, table_hbm, out_hbm, idx_blk, rows_v,
                      sem_g, sem_w, sem_i):
        wid = lax.axis_index("s") * NC + lax.axis_index("c")
        row_base = wid * n_slots

        def blk_copy(kb):
            return pltpu.make_async_copy(
                idx_hbm.at[pl.ds(row_base + kb * SPB, SPB)],
                idx_blk.at[lax.rem(kb, 2)], sem_i)

        def gath_copy(s, b):
            return pltpu.make_async_copy(
                table_hbm.at[idx_blk.at[lax.rem(s // SPB, 2), lax.rem(s, SPB)]],
                rows_v.at[b], sem_g.at[b])

        def out_copy(s, b):
            return pltpu.make_async_copy(
                rows_v.at[b],
                out_hbm.at[pl.ds((row_base + s) * CHUNK, CHUNK)],
                sem_w.at[b])

        pltpu.sync_copy(idx_hbm.at[pl.ds(row_base, SPB)], idx_blk.at[0])
        blk_copy(1).start()
        for b in range(2):
            gath_copy(b, b).start()

        def body(q, carry):
            for b4 in range(NBUF):
                s = NBUF * q + b4
                b = b4

                @pl.when(s >= 2)
                def _():
                    out_copy(s - 2, (b + 2) % NBUF).wait()

                sf = s + 2

                @pl.when(sf < n_slots)
                def _():
                    @pl.when(lax.rem(sf, SPB) == 0)
                    def _():
                        blk_copy(sf // SPB).wait()

                    gath_copy(sf, (b + 2) % NBUF).start()

                gath_copy(s, b).wait()
                out_copy(s, b).start()

                @pl.when((lax.rem(s, SPB) == SPB - 1)
                         & (s // SPB + 2 < n_blocks))
                def _():
                    blk_copy(s // SPB + 2).start()
            return carry

        lax.fori_loop(0, n_quads, body, 0)
        out_copy(n_slots - 2, (n_slots - 2) % NBUF).wait()
        out_copy(n_slots - 1, (n_slots - 1) % NBUF).wait()

    return gather_kernel


def kernel(idx, table):
    B_rows, H = idx.shape
    B = B_rows * H
    idx2d = idx.reshape(B // CHUNK, CHUNK)
    out = _make_gather(B)(idx2d, table)
    return out.reshape(B_rows, H, D)
